# baseline (speedup 1.0000x reference)
import functools

import jax
import jax.numpy as jnp
from jax import lax
from jax.experimental import pallas as pl
from jax.experimental.pallas import tpu as pltpu
from jax.experimental.pallas import tpu_sc as plsc

PADDING_IDX = 1
B = 4
S = 4096
D = 1024

NC = 2
NS = 16
NW = NC * NS

SPW = S // NW
CHUNK = 32
NCHUNK = SPW // CHUNK
LANES = 16
NGRP = SPW // LANES

_mesh = plsc.VectorSubcoreMesh(core_axis_name="c", subcore_axis_name="s")


@functools.partial(
    pl.kernel,
    out_type=jax.ShapeDtypeStruct((B * S, D), jnp.float32),
    mesh=_mesh,
    scratch_types=[
        pltpu.VMEM((B * SPW,), jnp.int32),
        pltpu.VMEM((B * SPW,), jnp.int32),
        pltpu.VMEM((SPW,), jnp.int32),
        pltpu.VMEM((2, CHUNK, D), jnp.float32),
        pltpu.VMEM((CHUNK, D), jnp.float32),
        pltpu.VMEM((B * NCHUNK,), jnp.int32),
        pltpu.SemaphoreType.DMA,
        pltpu.SemaphoreType.DMA,
        pltpu.SemaphoreType.DMA,
        pltpu.SemaphoreType.DMA,
        pltpu.SemaphoreType.DMA,
    ],
)
def _sc_embed(x_hbm, w_hbm, flag_hbm, cidx_hbm, out_hbm, x_v, idx_v, cidx_v,
              clean_v, fix_v, flag_v, xsem, g0, g1, s0_, s1_):
    gsem = (g0, g1)
    ssem = (s0_, s1_)

    wid = lax.axis_index("c") * NS + lax.axis_index("s")
    s0 = wid * SPW

    cidx_cp = pltpu.async_copy(cidx_hbm.at[pl.ds(s0, SPW)], cidx_v, xsem)
    flag_cp = pltpu.async_copy(flag_hbm.at[wid], flag_v, xsem)
    x_cps = [
        pltpu.async_copy(x_hbm.at[pl.ds(b * S + s0, SPW)],
                         x_v.at[pl.ds(b * SPW, SPW)], xsem)
        for b in range(B)
    ]

    def start_gather(c, sl):
        return pltpu.async_copy(
            w_hbm.at[cidx_v.at[pl.ds(c * CHUNK, CHUNK)]],
            clean_v.at[sl], gsem[sl])

    def start_scatters(c, sl):
        return [
            pltpu.async_copy(
                clean_v.at[sl],
                out_hbm.at[pl.ds(b * S + s0 + c * CHUNK, CHUNK)], ssem[sl])
            for b in range(B)
        ]

    cidx_cp.wait()
    g_cp = {0: start_gather(0, 0), 1: start_gather(1, 1)}
    flag_cp.wait()
    for cp in x_cps:
        cp.wait()

    iota = lax.broadcasted_iota(jnp.int32, (LANES,), 0)
    for b in range(B):
        for g in range(NGRP):
            tok = x_v[pl.ds(b * SPW + g * LANES, LANES)]
            pos = iota + (s0 + g * LANES + PADDING_IDX + 1)
            idx_v[pl.ds(b * SPW + g * LANES, LANES)] = jnp.where(
                tok == PADDING_IDX, PADDING_IDX, pos)

    s_cps = {}
    for c in range(NCHUNK):
        sl = c % 2
        if c >= 1 and c + 1 < NCHUNK:
            for cp in s_cps[sl ^ 1]:
                cp.wait()
            g_cp[sl ^ 1] = start_gather(c + 1, sl ^ 1)
        g_cp[sl].wait()
        s_cps[sl] = start_scatters(c, sl)
    for cp in s_cps[(NCHUNK - 2) % 2]:
        cp.wait()
    for cp in s_cps[(NCHUNK - 1) % 2]:
        cp.wait()

    plsc.subcore_barrier()

    flags = flag_v[pl.ds(0, B * NCHUNK)]
    for b in range(B):
        for c in range(NCHUNK):
            @pl.when(flags[b * NCHUNK + c] > 0)
            def _(b=b, c=c):
                pltpu.async_copy(
                    w_hbm.at[idx_v.at[pl.ds(b * SPW + c * CHUNK, CHUNK)]],
                    fix_v, xsem).wait()
                pltpu.sync_copy(
                    fix_v,
                    out_hbm.at[pl.ds(b * S + s0 + c * CHUNK, CHUNK)])


def kernel(X, weights):
    dirty = jnp.any(
        X.reshape(B, NW, NCHUNK, CHUNK) == PADDING_IDX, axis=-1)
    flags = dirty.transpose(1, 0, 2).reshape(NW, B * NCHUNK).astype(jnp.int32)
    cidx = jnp.arange(PADDING_IDX + 1, PADDING_IDX + 1 + S, dtype=jnp.int32)
    out = _sc_embed(X.reshape(B * S), weights, flags, cidx)
    return out.reshape(B, S, D)
